# baseline (speedup 1.0000x reference)
import jax
import jax.numpy as jnp
from jax.experimental import pallas as pl

_BLOCK = 1000


def _gates_kernel(x_ref, w_ref, b_ref, wco_ref, wlin_ref, blin_ref, o_ref):
    g = jnp.dot(x_ref[...], w_ref[...], preferred_element_type=jnp.float32)
    g = g + b_ref[...]
    i = jax.nn.sigmoid(g[:, 0:32])
    t = jnp.tanh(g[:, 32:64])
    c = i * t
    o = jax.nn.sigmoid(g[:, 64:96] + wco_ref[...] * c)
    h = jnp.maximum(o * jnp.tanh(c), 0.0)
    o_ref[...] = jnp.dot(h, wlin_ref[...], preferred_element_type=jnp.float32) + blin_ref[...]


def kernel(x, edge_index, edge_weight, W_x_i, b_x_i, W_h_i, b_h_i, b_i,
           W_x_f, b_x_f, W_h_f, b_h_f, b_f, W_x_c, b_x_c, W_h_c, b_h_c, b_c,
           W_x_o, b_x_o, W_h_o, b_h_o, b_o, w_c_i, w_c_f, w_c_o, W_lin, b_lin):
    n, f_in = x.shape
    f_out = W_x_i.shape[1]
    W = jnp.concatenate([W_x_i, W_x_c, W_x_o], axis=1)
    bias = jnp.concatenate([
        b_x_i + b_h_i + b_i[0],
        b_x_c + b_h_c + b_c[0],
        b_x_o + b_h_o + b_o[0],
    ]).reshape(1, 3 * f_out)
    blin = b_lin.reshape(1, 1)

    return pl.pallas_call(
        _gates_kernel,
        grid=(n // _BLOCK,),
        in_specs=[
            pl.BlockSpec((_BLOCK, f_in), lambda i: (i, 0)),
            pl.BlockSpec((f_in, 3 * f_out), lambda i: (0, 0)),
            pl.BlockSpec((1, 3 * f_out), lambda i: (0, 0)),
            pl.BlockSpec((1, f_out), lambda i: (0, 0)),
            pl.BlockSpec((f_out, 1), lambda i: (0, 0)),
            pl.BlockSpec((1, 1), lambda i: (0, 0)),
        ],
        out_specs=pl.BlockSpec((_BLOCK, 1), lambda i: (i, 0)),
        out_shape=jax.ShapeDtypeStruct((n, 1), jnp.float32),
    )(x, W, bias, w_c_o, W_lin, blin)


# speedup vs baseline: 1.1136x; 1.1136x over previous
import jax
import jax.numpy as jnp
from jax.experimental import pallas as pl

_BLOCK = 2000


def _gates_kernel(x_ref, w_ref, b_ref, wco_ref, wlin_ref, blin_ref, o_ref):
    g = jnp.dot(x_ref[...], w_ref[...], preferred_element_type=jnp.float32)
    g = g + b_ref[...]
    i = jax.nn.sigmoid(g[:, 0:32])
    t = jnp.tanh(g[:, 32:64])
    c = i * t
    o = jax.nn.sigmoid(g[:, 64:96] + wco_ref[...] * c)
    h = jnp.maximum(o * jnp.tanh(c), 0.0)
    o_ref[...] = jnp.dot(h, wlin_ref[...], preferred_element_type=jnp.float32) + blin_ref[...]


def kernel(x, edge_index, edge_weight, W_x_i, b_x_i, W_h_i, b_h_i, b_i,
           W_x_f, b_x_f, W_h_f, b_h_f, b_f, W_x_c, b_x_c, W_h_c, b_h_c, b_c,
           W_x_o, b_x_o, W_h_o, b_h_o, b_o, w_c_i, w_c_f, w_c_o, W_lin, b_lin):
    n, f_in = x.shape
    f_out = W_x_i.shape[1]
    W = jnp.concatenate([W_x_i, W_x_c, W_x_o], axis=1)
    bias = jnp.concatenate([
        b_x_i + b_h_i + b_i[0],
        b_x_c + b_h_c + b_c[0],
        b_x_o + b_h_o + b_o[0],
    ]).reshape(1, 3 * f_out)
    blin = b_lin.reshape(1, 1)

    return pl.pallas_call(
        _gates_kernel,
        grid=(n // _BLOCK,),
        in_specs=[
            pl.BlockSpec((_BLOCK, f_in), lambda i: (i, 0)),
            pl.BlockSpec((f_in, 3 * f_out), lambda i: (0, 0)),
            pl.BlockSpec((1, 3 * f_out), lambda i: (0, 0)),
            pl.BlockSpec((1, f_out), lambda i: (0, 0)),
            pl.BlockSpec((f_out, 1), lambda i: (0, 0)),
            pl.BlockSpec((1, 1), lambda i: (0, 0)),
        ],
        out_specs=pl.BlockSpec((_BLOCK, 1), lambda i: (i, 0)),
        out_shape=jax.ShapeDtypeStruct((n, 1), jnp.float32),
    )(x, W, bias, w_c_o, W_lin, blin)


# speedup vs baseline: 1.1166x; 1.0027x over previous
import jax
import jax.numpy as jnp
from jax.experimental import pallas as pl

_BLOCK = 5000


def _gates_kernel(x_ref, w_ref, b_ref, wco_ref, wlin_ref, blin_ref, o_ref):
    g = jnp.dot(x_ref[...], w_ref[...], preferred_element_type=jnp.float32)
    g = g + b_ref[...]
    i = jax.nn.sigmoid(g[:, 0:32])
    t = jnp.tanh(g[:, 32:64])
    c = i * t
    o = jax.nn.sigmoid(g[:, 64:96] + wco_ref[...] * c)
    h = jnp.maximum(o * jnp.tanh(c), 0.0)
    o_ref[...] = jnp.dot(h, wlin_ref[...], preferred_element_type=jnp.float32) + blin_ref[...]


def kernel(x, edge_index, edge_weight, W_x_i, b_x_i, W_h_i, b_h_i, b_i,
           W_x_f, b_x_f, W_h_f, b_h_f, b_f, W_x_c, b_x_c, W_h_c, b_h_c, b_c,
           W_x_o, b_x_o, W_h_o, b_h_o, b_o, w_c_i, w_c_f, w_c_o, W_lin, b_lin):
    n, f_in = x.shape
    f_out = W_x_i.shape[1]
    W = jnp.concatenate([W_x_i, W_x_c, W_x_o], axis=1)
    bias = jnp.concatenate([
        b_x_i + b_h_i + b_i[0],
        b_x_c + b_h_c + b_c[0],
        b_x_o + b_h_o + b_o[0],
    ]).reshape(1, 3 * f_out)
    blin = b_lin.reshape(1, 1)

    return pl.pallas_call(
        _gates_kernel,
        grid=(n // _BLOCK,),
        in_specs=[
            pl.BlockSpec((_BLOCK, f_in), lambda i: (i, 0)),
            pl.BlockSpec((f_in, 3 * f_out), lambda i: (0, 0)),
            pl.BlockSpec((1, 3 * f_out), lambda i: (0, 0)),
            pl.BlockSpec((1, f_out), lambda i: (0, 0)),
            pl.BlockSpec((f_out, 1), lambda i: (0, 0)),
            pl.BlockSpec((1, 1), lambda i: (0, 0)),
        ],
        out_specs=pl.BlockSpec((_BLOCK, 1), lambda i: (i, 0)),
        out_shape=jax.ShapeDtypeStruct((n, 1), jnp.float32),
    )(x, W, bias, w_c_o, W_lin, blin)
